# baseline (speedup 1.0000x reference)
import jax
import jax.numpy as jnp
from jax import lax
from jax.experimental import pallas as pl
from jax.experimental.pallas import tpu as pltpu
from jax.experimental.pallas import tpu_sc as plsc

D = 64
DW = 128
L = 50
PAIR = 2 * L
NC, NS = 2, 16
NW = NC * NS
LANES = 16
CH = 2
RING = 2
GRP = 8


def _lane_sum(v):
    parts = [v[i] for i in range(LANES)]
    while len(parts) > 1:
        parts = [parts[i] + parts[i + 1] for i in range(0, len(parts), 2)]
    return parts[0]


def _sc_body(table, src, dst, comb, s2dc, d2sc,
             out,
             idx_v, hpart_v, coef_v, srci_v, dsti_v,
             score_v, row0_v,
             buf0, buf1,
             psem, sem0, sem1):
    B = out.shape[0]
    epw = B // NW
    nch = epw // CH
    wid = lax.axis_index("s") * NC + lax.axis_index("c")
    e0 = wid * epw

    bufs = [buf0, buf1]
    sems = [sem0, sem1]

    pltpu.sync_copy(src.at[pl.ds(e0, epw)], srci_v)
    pltpu.sync_copy(dst.at[pl.ds(e0, epw)], dsti_v)
    pltpu.sync_copy(comb.at[pl.ds(wid * epw * PAIR, epw * PAIR)], idx_v)
    pltpu.sync_copy(s2dc.at[pl.ds(e0, epw)], coef_v.at[pl.ds(0, epw)])
    pltpu.sync_copy(d2sc.at[pl.ds(e0, epw)], coef_v.at[pl.ds(epw, epw)])
    pltpu.sync_copy(table.at[pl.ds(0, 1)], row0_v)

    prologue = [
        pltpu.make_async_copy(table.at[dsti_v], hpart_v.at[pl.ds(0, epw)], psem),
        pltpu.make_async_copy(table.at[srci_v], hpart_v.at[pl.ds(epw, epw)], psem),
    ]
    for cp in prologue:
        cp.start()

    def chunk_copy(c, p):
        return pltpu.make_async_copy(
            table.at[idx_v.at[pl.ds(CH * PAIR * c, CH * PAIR)]],
            bufs[p], sems[p])

    chunk_copy(0, 0).start()

    for cp in prologue:
        cp.wait()

    iota = lax.iota(jnp.int32, LANES)
    one = jnp.ones((LANES,), jnp.int32)
    m_lt2 = jnp.maximum(0, jnp.minimum(1, 2 - iota))
    m_ge2 = one - m_lt2
    m_ge12 = jnp.maximum(0, jnp.minimum(1, iota - 11))

    def zero_count_vecs(rowi):
        base = rowi * PAIR

        def zc(off):
            v = idx_v[pl.ds(base + off, LANES)]
            return one - jnp.minimum(jnp.abs(v), 1)

        z48 = zc(48)
        v_left = zc(0) + zc(16) + zc(32) + m_lt2 * z48
        v_right = m_ge2 * z48 + zc(64) + zc(80) + m_ge12 * zc(84)
        return v_left, v_right

    r0 = [row0_v[0, pl.ds(c * LANES, LANES)] for c in range(4)]
    onehots = [(one - jnp.minimum(jnp.abs(iota - i), 1)).astype(jnp.float32)
               for i in range(LANES)]

    def outer(g, carry):
        gb = g * LANES
        c1v = coef_v[pl.ds(gb, LANES)]
        c2v = coef_v[pl.ds(epw + gb, LANES)]
        c1sqv = c1v * c1v
        c2sqv = c2v * c2v
        csq1 = [c1sqv[i] for i in range(LANES)]
        csq2 = [c2sqv[i] for i in range(LANES)]
        sv = jnp.zeros((LANES,), jnp.float32)

        for k in range(GRP):
            c = g * GRP + k
            p = k % RING

            @pl.when(c + 1 < nch)
            def _():
                chunk_copy(c + 1, (k + 1) % RING).start()

            chunk_copy(c, p).wait()
            buf = bufs[p]

            rowA = CH * c
            rowB = rowA + 1
            vlA, vrA = zero_count_vecs(rowA)
            vlB, vrB = zero_count_vecs(rowB)
            packed = (vlA + (vrA << 6)) + ((vlB << 12) + (vrB << 18))
            tot = _lane_sum(packed)
            nA = (tot & 63, (tot >> 6) & 63)
            nB = ((tot >> 12) & 63, (tot >> 18) & 63)

            for e01 in range(2):
                e = 2 * c + e01
                rbase = e01 * L

                def row(l, accs):
                    r = rbase + l
                    new = []
                    for ch in range(4):
                        sl = pl.ds(ch * LANES, LANES)
                        new.append(accs[ch] + buf[r, sl])
                    for ch in range(4):
                        sl = pl.ds(ch * LANES, LANES)
                        new.append(accs[4 + ch] + buf[PAIR + r, sl])
                    return tuple(new)

                zeros = tuple(jnp.zeros((LANES,), jnp.float32)
                              for _ in range(8))
                accs = lax.fori_loop(0, L, row, zeros, unroll=5)

                naf = nA[e01].astype(jnp.float32)
                nbf = nB[e01].astype(jnp.float32)
                q = 2 * k + e01
                w = jnp.zeros((LANES,), jnp.float32)
                for ch in range(4):
                    sl = pl.ds(ch * LANES, LANES)
                    hd = hpart_v[e, sl]
                    hs = hpart_v[epw + e, sl]
                    accA = accs[ch] - naf * r0[ch]
                    accB = accs[4 + ch] - nbf * r0[ch]
                    w = (w + hd * (hs + csq1[q] * accA)
                         + (csq2[q] * hs) * accB)
                b1 = hpart_v[e, pl.ds(D, LANES)][0]
                b2 = hpart_v[epw + e, pl.ds(D, LANES)][0]
                sv = sv + onehots[q] * (_lane_sum(w) + b1 + b2)

        score_v[pl.ds(gb, LANES)] = sv
        return carry

    lax.fori_loop(0, nch // GRP, outer, 0)

    pltpu.sync_copy(score_v, out.at[pl.ds(e0, epw)])


@jax.jit
def kernel(h_output, node_biases, src, dst, s2d, s2dc, d2s, d2sc):
    B, Lx = s2d.shape
    N = h_output.shape[0]
    assert Lx == L and h_output.shape[1] == D and B % (NW * LANES) == 0

    table = jnp.concatenate(
        [h_output, node_biases[1:N + 1, None],
         jnp.zeros((N, DW - D - 1), jnp.float32)], axis=1)

    s2d_r = s2d.reshape(B * L // PAIR, PAIR)
    d2s_r = d2s.reshape(B * L // PAIR, PAIR)
    comb = jnp.stack([s2d_r, d2s_r], axis=1).reshape(B * PAIR)

    mesh = plsc.VectorSubcoreMesh(core_axis_name="c", subcore_axis_name="s",
                                  num_cores=NC, num_subcores=NS)
    epw = B // NW
    f = pl.kernel(
        _sc_body,
        out_type=jax.ShapeDtypeStruct((B,), jnp.float32),
        mesh=mesh,
        compiler_params=pltpu.CompilerParams(use_tc_tiling_on_sc=True),
        scratch_types=[
            pltpu.VMEM((epw * PAIR,), jnp.int32),
            pltpu.VMEM((2 * epw, DW), jnp.float32),
            pltpu.VMEM((2 * epw,), jnp.float32),
            pltpu.VMEM((epw,), jnp.int32),
            pltpu.VMEM((epw,), jnp.int32),
            pltpu.VMEM((epw,), jnp.float32),
            pltpu.VMEM((1, DW), jnp.float32),
            pltpu.VMEM((CH * PAIR, DW), jnp.float32),
            pltpu.VMEM((CH * PAIR, DW), jnp.float32),
            pltpu.SemaphoreType.DMA,
            pltpu.SemaphoreType.DMA,
            pltpu.SemaphoreType.DMA,
        ],
    )
    return f(table, src, dst, comb, s2dc, d2sc)


# speedup vs baseline: 2.2686x; 2.2686x over previous
import jax
import jax.numpy as jnp
from jax import lax
from jax.experimental import pallas as pl
from jax.experimental.pallas import tpu as pltpu
from jax.experimental.pallas import tpu_sc as plsc

D = 64
DW = 128
L = 50
PAIR = 2 * L
NC, NS = 2, 16
NW = NC * NS
LANES = 16
CH = 2
RING = 2
GRP = 8


def _lane_sum(v):
    parts = [v[i] for i in range(LANES)]
    while len(parts) > 1:
        parts = [parts[i] + parts[i + 1] for i in range(0, len(parts), 2)]
    return parts[0]


def _sc_body(table, src, dst, comb, s2dc, d2sc,
             out,
             idx_v, hpart_v, coef_v, srci_v, dsti_v,
             score_v, row0_v,
             buf0, buf1,
             psem, sem0, sem1):
    B = out.shape[0]
    epw = B // NW
    nch = epw // CH
    wid = lax.axis_index("s") * NC + lax.axis_index("c")
    e0 = wid * epw

    bufs = [buf0, buf1]
    sems = [sem0, sem1]

    pltpu.sync_copy(src.at[pl.ds(e0, epw)], srci_v)
    pltpu.sync_copy(dst.at[pl.ds(e0, epw)], dsti_v)
    pltpu.sync_copy(comb.at[pl.ds(wid * epw * PAIR, epw * PAIR)], idx_v)
    pltpu.sync_copy(s2dc.at[pl.ds(e0, epw)], coef_v.at[pl.ds(0, epw)])
    pltpu.sync_copy(d2sc.at[pl.ds(e0, epw)], coef_v.at[pl.ds(epw, epw)])
    pltpu.sync_copy(table.at[pl.ds(0, 1)], row0_v)

    prologue = [
        pltpu.make_async_copy(table.at[dsti_v], hpart_v.at[pl.ds(0, epw)], psem),
        pltpu.make_async_copy(table.at[srci_v], hpart_v.at[pl.ds(epw, epw)], psem),
    ]
    for cp in prologue:
        cp.start()

    def chunk_copy(c, p):
        return pltpu.make_async_copy(
            table.at[idx_v.at[pl.ds(CH * PAIR * c, CH * PAIR)]],
            bufs[p], sems[p])

    chunk_copy(0, 0).start()

    for cp in prologue:
        cp.wait()

    iota = lax.iota(jnp.int32, LANES)
    one = jnp.ones((LANES,), jnp.int32)
    m_lt2 = jnp.maximum(0, jnp.minimum(1, 2 - iota))
    m_ge2 = one - m_lt2
    m_ge12 = jnp.maximum(0, jnp.minimum(1, iota - 11))

    def zero_count_vecs(rowi):
        base = rowi * PAIR

        def zc(off):
            v = idx_v[pl.ds(base + off, LANES)]
            return one - jnp.minimum(jnp.abs(v), 1)

        z48 = zc(48)
        v_left = zc(0) + zc(16) + zc(32) + m_lt2 * z48
        v_right = m_ge2 * z48 + zc(64) + zc(80) + m_ge12 * zc(84)
        return v_left, v_right

    r0 = [row0_v[0, pl.ds(c * LANES, LANES)] for c in range(4)]
    onehots = [(one - jnp.minimum(jnp.abs(iota - i), 1)).astype(jnp.float32)
               for i in range(LANES)]

    def outer(g, carry):
        gb = g * LANES
        c1v = coef_v[pl.ds(gb, LANES)]
        c2v = coef_v[pl.ds(epw + gb, LANES)]
        c1sqv = c1v * c1v
        c2sqv = c2v * c2v
        csq1 = [c1sqv[i] for i in range(LANES)]
        csq2 = [c2sqv[i] for i in range(LANES)]
        sv = jnp.zeros((LANES,), jnp.float32)

        for k in range(GRP):
            c = g * GRP + k
            p = k % RING

            @pl.when(c + 1 < nch)
            def _():
                chunk_copy(c + 1, (k + 1) % RING).start()

            chunk_copy(c, p).wait()
            buf = bufs[p]

            rowA = CH * c
            rowB = rowA + 1
            vlA, vrA = zero_count_vecs(rowA)
            vlB, vrB = zero_count_vecs(rowB)
            packed = (vlA + (vrA << 6)) + ((vlB << 12) + (vrB << 18))
            tot = _lane_sum(packed)
            nA = (tot & 63, (tot >> 6) & 63)
            nB = ((tot >> 12) & 63, (tot >> 18) & 63)

            for e01 in range(2):
                e = 2 * c + e01
                rbase = e01 * L

                def row(l, accs):
                    r = rbase + l
                    new = []
                    for ch in range(4):
                        sl = pl.ds(ch * LANES, LANES)
                        new.append(accs[ch] + buf[r, sl])
                    for ch in range(4):
                        sl = pl.ds(ch * LANES, LANES)
                        new.append(accs[4 + ch] + buf[PAIR + r, sl])
                    return tuple(new)

                zeros = tuple(jnp.zeros((LANES,), jnp.float32)
                              for _ in range(8))
                accs = lax.fori_loop(0, L, row, zeros, unroll=5)

                naf = nA[e01].astype(jnp.float32)
                nbf = nB[e01].astype(jnp.float32)
                q = 2 * k + e01
                w = jnp.zeros((LANES,), jnp.float32)
                for ch in range(4):
                    sl = pl.ds(ch * LANES, LANES)
                    hd = hpart_v[e, sl]
                    hs = hpart_v[epw + e, sl]
                    accA = accs[ch] - naf * r0[ch]
                    accB = accs[4 + ch] - nbf * r0[ch]
                    w = (w + hd * (hs + csq1[q] * accA)
                         + (csq2[q] * hs) * accB)
                b1 = hpart_v[e, pl.ds(D, LANES)][0]
                b2 = hpart_v[epw + e, pl.ds(D, LANES)][0]
                sv = sv + onehots[q] * (_lane_sum(w) + b1 + b2)

        score_v[pl.ds(gb, LANES)] = sv
        return carry

    lax.fori_loop(0, nch // GRP, outer, 0)

    pltpu.sync_copy(score_v, out.at[pl.ds(e0, epw)])


def _tc_pack_body(ht_ref, bias_ref, out_ref):
    xt = jnp.transpose(ht_ref[...], (1, 0))
    bk = xt.shape[0]
    bcol = bias_ref[...].reshape(bk, 1)
    pad = jnp.zeros((bk, DW - D - 1), jnp.float32)
    out_ref[...] = jnp.concatenate([xt, bcol, pad], axis=1)


def _tc_pack(h_output, node_biases):
    N = h_output.shape[0]
    BK = 8192
    ht = h_output.T
    bias = node_biases[1:N + 1]
    return pl.pallas_call(
        _tc_pack_body,
        grid=((N + BK - 1) // BK,),
        in_specs=[
            pl.BlockSpec((D, BK), lambda i: (0, i)),
            pl.BlockSpec((BK,), lambda i: (i,)),
        ],
        out_specs=pl.BlockSpec((BK, DW), lambda i: (i, 0)),
        out_shape=jax.ShapeDtypeStruct((N, DW), jnp.float32),
    )(ht, bias)


@jax.jit
def kernel(h_output, node_biases, src, dst, s2d, s2dc, d2s, d2sc):
    B, Lx = s2d.shape
    N = h_output.shape[0]
    assert Lx == L and h_output.shape[1] == D and B % (NW * LANES) == 0

    table = _tc_pack(h_output, node_biases)

    s2d_r = s2d.reshape(B * L // PAIR, PAIR)
    d2s_r = d2s.reshape(B * L // PAIR, PAIR)
    comb = jnp.stack([s2d_r, d2s_r], axis=1).reshape(B * PAIR)

    mesh = plsc.VectorSubcoreMesh(core_axis_name="c", subcore_axis_name="s",
                                  num_cores=NC, num_subcores=NS)
    epw = B // NW
    f = pl.kernel(
        _sc_body,
        out_type=jax.ShapeDtypeStruct((B,), jnp.float32),
        mesh=mesh,
        compiler_params=pltpu.CompilerParams(use_tc_tiling_on_sc=True),
        scratch_types=[
            pltpu.VMEM((epw * PAIR,), jnp.int32),
            pltpu.VMEM((2 * epw, DW), jnp.float32),
            pltpu.VMEM((2 * epw,), jnp.float32),
            pltpu.VMEM((epw,), jnp.int32),
            pltpu.VMEM((epw,), jnp.int32),
            pltpu.VMEM((epw,), jnp.float32),
            pltpu.VMEM((1, DW), jnp.float32),
            pltpu.VMEM((CH * PAIR, DW), jnp.float32),
            pltpu.VMEM((CH * PAIR, DW), jnp.float32),
            pltpu.SemaphoreType.DMA,
            pltpu.SemaphoreType.DMA,
            pltpu.SemaphoreType.DMA,
        ],
    )
    return f(table, src, dst, comb, s2dc, d2sc)


# speedup vs baseline: 2.4499x; 1.0799x over previous
import jax
import jax.numpy as jnp
from jax import lax
from jax.experimental import pallas as pl
from jax.experimental.pallas import tpu as pltpu
from jax.experimental.pallas import tpu_sc as plsc

D = 64
DW = 128
L = 50
PAIR = 2 * L
NC, NS = 2, 16
NW = NC * NS
LANES = 16
CH = 2
RING = 2
GRP = 8


def _lane_sum(v):
    parts = [v[i] for i in range(LANES)]
    while len(parts) > 1:
        parts = [parts[i] + parts[i + 1] for i in range(0, len(parts), 2)]
    return parts[0]


def _sc_body(table, src, dst, comb, s2dc, d2sc,
             out,
             idx_v, hpart_v, coef_v, srci_v, dsti_v,
             score_v, row0_v,
             buf0, buf1,
             psem, sem0, sem1):
    B = out.shape[0]
    epw = B // NW
    nch = epw // CH
    wid = lax.axis_index("s") * NC + lax.axis_index("c")
    e0 = wid * epw

    bufs = [buf0, buf1]
    sems = [sem0, sem1]

    pltpu.sync_copy(src.at[pl.ds(e0, epw)], srci_v)
    pltpu.sync_copy(dst.at[pl.ds(e0, epw)], dsti_v)
    pltpu.sync_copy(comb.at[pl.ds(wid * epw * PAIR, epw * PAIR)], idx_v)
    pltpu.sync_copy(s2dc.at[pl.ds(e0, epw)], coef_v.at[pl.ds(0, epw)])
    pltpu.sync_copy(d2sc.at[pl.ds(e0, epw)], coef_v.at[pl.ds(epw, epw)])
    pltpu.sync_copy(table.at[pl.ds(0, 1)], row0_v)

    prologue = [
        pltpu.make_async_copy(table.at[dsti_v], hpart_v.at[pl.ds(0, epw)], psem),
        pltpu.make_async_copy(table.at[srci_v], hpart_v.at[pl.ds(epw, epw)], psem),
    ]
    for cp in prologue:
        cp.start()

    def chunk_copy(c, p):
        return pltpu.make_async_copy(
            table.at[idx_v.at[pl.ds(CH * PAIR * c, CH * PAIR)]],
            bufs[p], sems[p])

    chunk_copy(0, 0).start()

    for cp in prologue:
        cp.wait()

    iota = lax.iota(jnp.int32, LANES)
    one = jnp.ones((LANES,), jnp.int32)
    m_lt2 = jnp.maximum(0, jnp.minimum(1, 2 - iota))
    m_ge2 = one - m_lt2
    m_ge12 = jnp.maximum(0, jnp.minimum(1, iota - 11))

    def zero_count_vecs(rowi):
        base = rowi * PAIR

        def zc(off):
            v = idx_v[pl.ds(base + off, LANES)]
            return one - jnp.minimum(jnp.abs(v), 1)

        z48 = zc(48)
        v_left = zc(0) + zc(16) + zc(32) + m_lt2 * z48
        v_right = m_ge2 * z48 + zc(64) + zc(80) + m_ge12 * zc(84)
        return v_left, v_right

    r0 = [row0_v[0, pl.ds(c * LANES, LANES)] for c in range(4)]
    onehots = [(one - jnp.minimum(jnp.abs(iota - i), 1)).astype(jnp.float32)
               for i in range(LANES)]

    def outer(g, carry):
        gb = g * LANES
        c1v = coef_v[pl.ds(gb, LANES)]
        c2v = coef_v[pl.ds(epw + gb, LANES)]
        c1sqv = c1v * c1v
        c2sqv = c2v * c2v
        csq1 = [c1sqv[i] for i in range(LANES)]
        csq2 = [c2sqv[i] for i in range(LANES)]
        sv = jnp.zeros((LANES,), jnp.float32)

        for k in range(GRP):
            c = g * GRP + k
            p = k % RING

            @pl.when(c + 1 < nch)
            def _():
                chunk_copy(c + 1, (k + 1) % RING).start()

            chunk_copy(c, p).wait()
            buf = bufs[p]

            rowA = CH * c
            rowB = rowA + 1
            vlA, vrA = zero_count_vecs(rowA)
            vlB, vrB = zero_count_vecs(rowB)
            packed = (vlA + (vrA << 6)) + ((vlB << 12) + (vrB << 18))
            tot = _lane_sum(packed)
            nA = (tot & 63, (tot >> 6) & 63)
            nB = ((tot >> 12) & 63, (tot >> 18) & 63)

            for e01 in range(2):
                e = 2 * c + e01
                rbase = e01 * L

                def row(l, accs):
                    r = rbase + l
                    new = []
                    for ch in range(4):
                        sl = pl.ds(ch * LANES, LANES)
                        new.append(accs[ch] + buf[r, sl])
                    for ch in range(4):
                        sl = pl.ds(ch * LANES, LANES)
                        new.append(accs[4 + ch] + buf[PAIR + r, sl])
                    return tuple(new)

                zeros = tuple(jnp.zeros((LANES,), jnp.float32)
                              for _ in range(8))
                accs = lax.fori_loop(0, L, row, zeros, unroll=5)

                naf = nA[e01].astype(jnp.float32)
                nbf = nB[e01].astype(jnp.float32)
                q = 2 * k + e01
                w = jnp.zeros((LANES,), jnp.float32)
                for ch in range(4):
                    sl = pl.ds(ch * LANES, LANES)
                    hd = hpart_v[e, sl]
                    hs = hpart_v[epw + e, sl]
                    accA = accs[ch] - naf * r0[ch]
                    accB = accs[4 + ch] - nbf * r0[ch]
                    w = (w + hd * (hs + csq1[q] * accA)
                         + (csq2[q] * hs) * accB)
                b1 = hpart_v[e, pl.ds(D, LANES)][0]
                b2 = hpart_v[epw + e, pl.ds(D, LANES)][0]
                sv = sv + onehots[q] * (_lane_sum(w) + b1 + b2)

        score_v[pl.ds(gb, LANES)] = sv
        return carry

    lax.fori_loop(0, nch // GRP, outer, 0)

    pltpu.sync_copy(score_v, out.at[pl.ds(e0, epw)])


def _tc_pack_body(ht_ref, bias_ref, out_ref):
    xt = jnp.transpose(ht_ref[...], (1, 0))
    bk = xt.shape[0]
    bcol = bias_ref[...].reshape(bk, 1)
    pad = jnp.zeros((bk, DW - D - 1), jnp.float32)
    out_ref[...] = jnp.concatenate([xt, bcol, pad], axis=1)


def _tc_pack(h_output, node_biases):
    N = h_output.shape[0]
    BK = 16384
    ht = h_output.T
    bias = node_biases[1:N + 1]
    return pl.pallas_call(
        _tc_pack_body,
        grid=((N + BK - 1) // BK,),
        in_specs=[
            pl.BlockSpec((D, BK), lambda i: (0, i)),
            pl.BlockSpec((BK,), lambda i: (i,)),
        ],
        out_specs=pl.BlockSpec((BK, DW), lambda i: (i, 0)),
        out_shape=jax.ShapeDtypeStruct((N, DW), jnp.float32),
    )(ht, bias)


@jax.jit
def kernel(h_output, node_biases, src, dst, s2d, s2dc, d2s, d2sc):
    B, Lx = s2d.shape
    N = h_output.shape[0]
    assert Lx == L and h_output.shape[1] == D and B % (NW * LANES) == 0

    table = _tc_pack(h_output, node_biases)

    s2d_r = s2d.reshape(B * L // PAIR, PAIR)
    d2s_r = d2s.reshape(B * L // PAIR, PAIR)
    comb = jnp.stack([s2d_r, d2s_r], axis=1).reshape(B * PAIR)

    mesh = plsc.VectorSubcoreMesh(core_axis_name="c", subcore_axis_name="s",
                                  num_cores=NC, num_subcores=NS)
    epw = B // NW
    f = pl.kernel(
        _sc_body,
        out_type=jax.ShapeDtypeStruct((B,), jnp.float32),
        mesh=mesh,
        compiler_params=pltpu.CompilerParams(use_tc_tiling_on_sc=True),
        scratch_types=[
            pltpu.VMEM((epw * PAIR,), jnp.int32),
            pltpu.VMEM((2 * epw, DW), jnp.float32),
            pltpu.VMEM((2 * epw,), jnp.float32),
            pltpu.VMEM((epw,), jnp.int32),
            pltpu.VMEM((epw,), jnp.int32),
            pltpu.VMEM((epw,), jnp.float32),
            pltpu.VMEM((1, DW), jnp.float32),
            pltpu.VMEM((CH * PAIR, DW), jnp.float32),
            pltpu.VMEM((CH * PAIR, DW), jnp.float32),
            pltpu.SemaphoreType.DMA,
            pltpu.SemaphoreType.DMA,
            pltpu.SemaphoreType.DMA,
        ],
    )
    return f(table, src, dst, comb, s2dc, d2sc)


# speedup vs baseline: 2.5235x; 1.0300x over previous
import jax
import jax.numpy as jnp
from jax import lax
from jax.experimental import pallas as pl
from jax.experimental.pallas import tpu as pltpu
from jax.experimental.pallas import tpu_sc as plsc

D = 64
DW = 128
L = 50
PAIR = 2 * L
NC, NS = 2, 16
NW = NC * NS
LANES = 16
CH = 2
RING = 2
GRP = 8


def _lane_sum(v):
    parts = [v[i] for i in range(LANES)]
    while len(parts) > 1:
        parts = [parts[i] + parts[i + 1] for i in range(0, len(parts), 2)]
    return parts[0]


def _sc_body(table, src, dst, comb, s2dc, d2sc,
             out,
             idx_v, hpart_v, coef_v, srci_v, dsti_v,
             score_v, row0_v,
             buf0, buf1,
             psem, sem0, sem1):
    B = out.shape[0]
    epw = B // NW
    nch = epw // CH
    wid = lax.axis_index("s") * NC + lax.axis_index("c")
    e0 = wid * epw

    bufs = [buf0, buf1]
    sems = [sem0, sem1]

    pltpu.sync_copy(src.at[pl.ds(e0, epw)], srci_v)
    pltpu.sync_copy(dst.at[pl.ds(e0, epw)], dsti_v)
    pltpu.sync_copy(comb.at[pl.ds(wid * epw * PAIR, epw * PAIR)], idx_v)
    pltpu.sync_copy(s2dc.at[pl.ds(e0, epw)], coef_v.at[pl.ds(0, epw)])
    pltpu.sync_copy(d2sc.at[pl.ds(e0, epw)], coef_v.at[pl.ds(epw, epw)])
    pltpu.sync_copy(table.at[pl.ds(0, 1)], row0_v)

    prologue = [
        pltpu.make_async_copy(table.at[dsti_v], hpart_v.at[pl.ds(0, epw)], psem),
        pltpu.make_async_copy(table.at[srci_v], hpart_v.at[pl.ds(epw, epw)], psem),
    ]
    for cp in prologue:
        cp.start()

    def chunk_copy(c, p):
        return pltpu.make_async_copy(
            table.at[idx_v.at[pl.ds(CH * PAIR * c, CH * PAIR)]],
            bufs[p], sems[p])

    chunk_copy(0, 0).start()

    for cp in prologue:
        cp.wait()

    iota = lax.iota(jnp.int32, LANES)
    one = jnp.ones((LANES,), jnp.int32)
    m_lt2 = jnp.maximum(0, jnp.minimum(1, 2 - iota))
    m_ge2 = one - m_lt2
    m_ge12 = jnp.maximum(0, jnp.minimum(1, iota - 11))

    def zero_count_vecs(rowi):
        base = rowi * PAIR

        def zc(off):
            v = idx_v[pl.ds(base + off, LANES)]
            return one - jnp.minimum(jnp.abs(v), 1)

        z48 = zc(48)
        v_left = zc(0) + zc(16) + zc(32) + m_lt2 * z48
        v_right = m_ge2 * z48 + zc(64) + zc(80) + m_ge12 * zc(84)
        return v_left, v_right

    r0 = [row0_v[0, pl.ds(c * LANES, LANES)] for c in range(4)]
    onehots = [(one - jnp.minimum(jnp.abs(iota - i), 1)).astype(jnp.float32)
               for i in range(LANES)]

    def outer(g, carry):
        gb = g * LANES
        c1v = coef_v[pl.ds(gb, LANES)]
        c2v = coef_v[pl.ds(epw + gb, LANES)]
        c1sqv = c1v * c1v
        c2sqv = c2v * c2v
        csq1 = [c1sqv[i] for i in range(LANES)]
        csq2 = [c2sqv[i] for i in range(LANES)]
        sv = jnp.zeros((LANES,), jnp.float32)

        for k in range(GRP):
            c = g * GRP + k
            p = k % RING

            @pl.when(c + 1 < nch)
            def _():
                chunk_copy(c + 1, (k + 1) % RING).start()

            chunk_copy(c, p).wait()
            buf = bufs[p]

            rowA = CH * c
            rowB = rowA + 1
            vlA, vrA = zero_count_vecs(rowA)
            vlB, vrB = zero_count_vecs(rowB)
            packed = (vlA + (vrA << 6)) + ((vlB << 12) + (vrB << 18))
            tot = _lane_sum(packed)
            nA = (tot & 63, (tot >> 6) & 63)
            nB = ((tot >> 12) & 63, (tot >> 18) & 63)

            for e01 in range(2):
                e = 2 * c + e01
                rbase = e01 * L

                def row(l, accs):
                    r = rbase + l
                    new = []
                    for ch in range(4):
                        sl = pl.ds(ch * LANES, LANES)
                        new.append(accs[ch] + buf[r, sl])
                    for ch in range(4):
                        sl = pl.ds(ch * LANES, LANES)
                        new.append(accs[4 + ch] + buf[PAIR + r, sl])
                    return tuple(new)

                zeros = tuple(jnp.zeros((LANES,), jnp.float32)
                              for _ in range(8))
                accs = lax.fori_loop(0, L, row, zeros, unroll=5)

                naf = nA[e01].astype(jnp.float32)
                nbf = nB[e01].astype(jnp.float32)
                q = 2 * k + e01
                w = jnp.zeros((LANES,), jnp.float32)
                for ch in range(4):
                    sl = pl.ds(ch * LANES, LANES)
                    hd = hpart_v[e, sl]
                    hs = hpart_v[epw + e, sl]
                    accA = accs[ch] - naf * r0[ch]
                    accB = accs[4 + ch] - nbf * r0[ch]
                    w = (w + hd * (hs + csq1[q] * accA)
                         + (csq2[q] * hs) * accB)
                b1 = hpart_v[e, pl.ds(D, LANES)][0]
                b2 = hpart_v[epw + e, pl.ds(D, LANES)][0]
                sv = sv + onehots[q] * (_lane_sum(w) + b1 + b2)

        score_v[pl.ds(gb, LANES)] = sv
        return carry

    lax.fori_loop(0, nch // GRP, outer, 0)

    pltpu.sync_copy(score_v, out.at[pl.ds(e0, epw)])


def _tc_pack_body(ht_ref, bias_ref, out_ref):
    xt = jnp.transpose(ht_ref[...], (1, 0))
    bk = xt.shape[0]
    bcol = bias_ref[...].reshape(bk, 1)
    pad = jnp.zeros((bk, DW - D - 1), jnp.float32)
    out_ref[...] = jnp.concatenate([xt, bcol, pad], axis=1)


def _tc_pack(h_output, node_biases):
    N = h_output.shape[0]
    BK = 24576
    ht = h_output.T
    bias = node_biases[1:N + 1]
    return pl.pallas_call(
        _tc_pack_body,
        grid=((N + BK - 1) // BK,),
        in_specs=[
            pl.BlockSpec((D, BK), lambda i: (0, i)),
            pl.BlockSpec((BK,), lambda i: (i,)),
        ],
        out_specs=pl.BlockSpec((BK, DW), lambda i: (i, 0)),
        out_shape=jax.ShapeDtypeStruct((N, DW), jnp.float32),
    )(ht, bias)


@jax.jit
def kernel(h_output, node_biases, src, dst, s2d, s2dc, d2s, d2sc):
    B, Lx = s2d.shape
    N = h_output.shape[0]
    assert Lx == L and h_output.shape[1] == D and B % (NW * LANES) == 0

    table = _tc_pack(h_output, node_biases)

    s2d_r = s2d.reshape(B * L // PAIR, PAIR)
    d2s_r = d2s.reshape(B * L // PAIR, PAIR)
    comb = jnp.stack([s2d_r, d2s_r], axis=1).reshape(B * PAIR)

    mesh = plsc.VectorSubcoreMesh(core_axis_name="c", subcore_axis_name="s",
                                  num_cores=NC, num_subcores=NS)
    epw = B // NW
    f = pl.kernel(
        _sc_body,
        out_type=jax.ShapeDtypeStruct((B,), jnp.float32),
        mesh=mesh,
        compiler_params=pltpu.CompilerParams(use_tc_tiling_on_sc=True),
        scratch_types=[
            pltpu.VMEM((epw * PAIR,), jnp.int32),
            pltpu.VMEM((2 * epw, DW), jnp.float32),
            pltpu.VMEM((2 * epw,), jnp.float32),
            pltpu.VMEM((epw,), jnp.int32),
            pltpu.VMEM((epw,), jnp.int32),
            pltpu.VMEM((epw,), jnp.float32),
            pltpu.VMEM((1, DW), jnp.float32),
            pltpu.VMEM((CH * PAIR, DW), jnp.float32),
            pltpu.VMEM((CH * PAIR, DW), jnp.float32),
            pltpu.SemaphoreType.DMA,
            pltpu.SemaphoreType.DMA,
            pltpu.SemaphoreType.DMA,
        ],
    )
    return f(table, src, dst, comb, s2dc, d2sc)


# speedup vs baseline: 2.5269x; 1.0014x over previous
import jax
import jax.numpy as jnp
from jax import lax
from jax.experimental import pallas as pl
from jax.experimental.pallas import tpu as pltpu
from jax.experimental.pallas import tpu_sc as plsc

D = 64
DW = 128
L = 50
PAIR = 2 * L
NC, NS = 2, 16
NW = NC * NS
LANES = 16
CH = 2
RING = 2
GRP = 8


def _lane_sum(v):
    parts = [v[i] for i in range(LANES)]
    while len(parts) > 1:
        parts = [parts[i] + parts[i + 1] for i in range(0, len(parts), 2)]
    return parts[0]


def _sc_body(table, src, dst, comb, s2dc, d2sc,
             out,
             idx_v, hpart_v, coef_v, srci_v, dsti_v,
             score_v, row0_v,
             buf0, buf1,
             psem, sem0, sem1):
    B = out.shape[0]
    epw = B // NW
    nch = epw // CH
    wid = lax.axis_index("s") * NC + lax.axis_index("c")
    e0 = wid * epw

    bufs = [buf0, buf1]
    sems = [sem0, sem1]

    pltpu.sync_copy(src.at[pl.ds(e0, epw)], srci_v)
    pltpu.sync_copy(dst.at[pl.ds(e0, epw)], dsti_v)
    pltpu.sync_copy(comb.at[pl.ds(wid * epw * PAIR, epw * PAIR)], idx_v)
    pltpu.sync_copy(s2dc.at[pl.ds(e0, epw)], coef_v.at[pl.ds(0, epw)])
    pltpu.sync_copy(d2sc.at[pl.ds(e0, epw)], coef_v.at[pl.ds(epw, epw)])
    pltpu.sync_copy(table.at[pl.ds(0, 1)], row0_v)

    prologue = [
        pltpu.make_async_copy(table.at[dsti_v], hpart_v.at[pl.ds(0, epw)], psem),
        pltpu.make_async_copy(table.at[srci_v], hpart_v.at[pl.ds(epw, epw)], psem),
    ]
    for cp in prologue:
        cp.start()

    def chunk_copy(c, p):
        return pltpu.make_async_copy(
            table.at[idx_v.at[pl.ds(CH * PAIR * c, CH * PAIR)]],
            bufs[p], sems[p])

    chunk_copy(0, 0).start()

    for cp in prologue:
        cp.wait()

    iota = lax.iota(jnp.int32, LANES)
    one = jnp.ones((LANES,), jnp.int32)
    m_lt2 = jnp.maximum(0, jnp.minimum(1, 2 - iota))
    m_ge2 = one - m_lt2
    m_ge12 = jnp.maximum(0, jnp.minimum(1, iota - 11))

    def zero_count_vecs(rowi):
        base = rowi * PAIR

        def zc(off):
            v = idx_v[pl.ds(base + off, LANES)]
            return one - jnp.minimum(jnp.abs(v), 1)

        z48 = zc(48)
        v_left = zc(0) + zc(16) + zc(32) + m_lt2 * z48
        v_right = m_ge2 * z48 + zc(64) + zc(80) + m_ge12 * zc(84)
        return v_left, v_right

    r0 = [row0_v[0, pl.ds(c * LANES, LANES)] for c in range(4)]
    onehots = [(one - jnp.minimum(jnp.abs(iota - i), 1)).astype(jnp.float32)
               for i in range(LANES)]

    def outer(g, carry):
        gb = g * LANES
        c1v = coef_v[pl.ds(gb, LANES)]
        c2v = coef_v[pl.ds(epw + gb, LANES)]
        c1sqv = c1v * c1v
        c2sqv = c2v * c2v
        csq1 = [c1sqv[i] for i in range(LANES)]
        csq2 = [c2sqv[i] for i in range(LANES)]
        sv = jnp.zeros((LANES,), jnp.float32)

        for k in range(GRP):
            c = g * GRP + k
            p = k % RING

            @pl.when(c + 1 < nch)
            def _():
                chunk_copy(c + 1, (k + 1) % RING).start()

            chunk_copy(c, p).wait()
            buf = bufs[p]

            rowA = CH * c
            rowB = rowA + 1
            vlA, vrA = zero_count_vecs(rowA)
            vlB, vrB = zero_count_vecs(rowB)
            packed = (vlA + (vrA << 6)) + ((vlB << 12) + (vrB << 18))
            tot = _lane_sum(packed)
            nA = (tot & 63, (tot >> 6) & 63)
            nB = ((tot >> 12) & 63, (tot >> 18) & 63)

            for e01 in range(2):
                e = 2 * c + e01
                rbase = e01 * L

                def row(l, accs):
                    r = rbase + l
                    new = []
                    for ch in range(4):
                        sl = pl.ds(ch * LANES, LANES)
                        new.append(accs[ch] + buf[r, sl])
                    for ch in range(4):
                        sl = pl.ds(ch * LANES, LANES)
                        new.append(accs[4 + ch] + buf[PAIR + r, sl])
                    return tuple(new)

                zeros = tuple(jnp.zeros((LANES,), jnp.float32)
                              for _ in range(8))
                accs = lax.fori_loop(0, L, row, zeros, unroll=5)

                naf = nA[e01].astype(jnp.float32)
                nbf = nB[e01].astype(jnp.float32)
                q = 2 * k + e01
                w = jnp.zeros((LANES,), jnp.float32)
                for ch in range(4):
                    sl = pl.ds(ch * LANES, LANES)
                    hd = hpart_v[e, sl]
                    hs = hpart_v[epw + e, sl]
                    accA = accs[ch] - naf * r0[ch]
                    accB = accs[4 + ch] - nbf * r0[ch]
                    w = (w + hd * (hs + csq1[q] * accA)
                         + (csq2[q] * hs) * accB)
                b1 = hpart_v[e, pl.ds(D, LANES)][0]
                b2 = hpart_v[epw + e, pl.ds(D, LANES)][0]
                sv = sv + onehots[q] * (_lane_sum(w) + b1 + b2)

        score_v[pl.ds(gb, LANES)] = sv
        return carry

    lax.fori_loop(0, nch // GRP, outer, 0)

    pltpu.sync_copy(score_v, out.at[pl.ds(e0, epw)])


def _tc_pack_body(ht_ref, bias_ref, out_ref):
    xt = jnp.transpose(ht_ref[...], (1, 0))
    bk = xt.shape[0]
    bcol = bias_ref[...].reshape(bk, 1)
    pad = jnp.zeros((bk, DW - D - 1), jnp.float32)
    out_ref[...] = jnp.concatenate([xt, bcol, pad], axis=1)


def _tc_pack(h_output, node_biases):
    N = h_output.shape[0]
    BK = 28672
    ht = h_output.T
    bias = node_biases[1:N + 1]
    return pl.pallas_call(
        _tc_pack_body,
        grid=((N + BK - 1) // BK,),
        in_specs=[
            pl.BlockSpec((D, BK), lambda i: (0, i)),
            pl.BlockSpec((BK,), lambda i: (i,)),
        ],
        out_specs=pl.BlockSpec((BK, DW), lambda i: (i, 0)),
        out_shape=jax.ShapeDtypeStruct((N, DW), jnp.float32),
    )(ht, bias)


@jax.jit
def kernel(h_output, node_biases, src, dst, s2d, s2dc, d2s, d2sc):
    B, Lx = s2d.shape
    N = h_output.shape[0]
    assert Lx == L and h_output.shape[1] == D and B % (NW * LANES) == 0

    table = _tc_pack(h_output, node_biases)

    s2d_r = s2d.reshape(B * L // PAIR, PAIR)
    d2s_r = d2s.reshape(B * L // PAIR, PAIR)
    comb = jnp.stack([s2d_r, d2s_r], axis=1).reshape(B * PAIR)

    mesh = plsc.VectorSubcoreMesh(core_axis_name="c", subcore_axis_name="s",
                                  num_cores=NC, num_subcores=NS)
    epw = B // NW
    f = pl.kernel(
        _sc_body,
        out_type=jax.ShapeDtypeStruct((B,), jnp.float32),
        mesh=mesh,
        compiler_params=pltpu.CompilerParams(use_tc_tiling_on_sc=True),
        scratch_types=[
            pltpu.VMEM((epw * PAIR,), jnp.int32),
            pltpu.VMEM((2 * epw, DW), jnp.float32),
            pltpu.VMEM((2 * epw,), jnp.float32),
            pltpu.VMEM((epw,), jnp.int32),
            pltpu.VMEM((epw,), jnp.int32),
            pltpu.VMEM((epw,), jnp.float32),
            pltpu.VMEM((1, DW), jnp.float32),
            pltpu.VMEM((CH * PAIR, DW), jnp.float32),
            pltpu.VMEM((CH * PAIR, DW), jnp.float32),
            pltpu.SemaphoreType.DMA,
            pltpu.SemaphoreType.DMA,
            pltpu.SemaphoreType.DMA,
        ],
    )
    return f(table, src, dst, comb, s2dc, d2sc)
